# Initial kernel scaffold; baseline (speedup 1.0000x reference)
import jax
import jax.numpy as jnp
from jax.experimental import pallas as pl


def kernel(LL, LH, HL, HH, conv_w, conv_b, bn_gamma, bn_beta):
    raise NotImplementedError("write your pallas kernel here")


# speedup vs baseline: 1.1024x; 1.1024x over previous
import functools

import jax
import jax.numpy as jnp
from jax.experimental import pallas as pl
from jax.experimental.pallas import tpu as pltpu

_f32 = jnp.float32
_bf16 = jnp.bfloat16

_TAPS = [(dy, dx) for dy in (-1, 0, 1) for dx in (-1, 0, 1)]


def _upconv_kernel(ll_ref, lh_ref, hl_ref, hh_ref, w_ref, b_ref,
                   y_ref, stats_ref, a_sc, *, th):
    H, W, Cin = ll_ref.shape[1], ll_ref.shape[2], ll_ref.shape[3]
    Cout = w_ref.shape[1]

    r = pl.program_id(1)
    nrb = pl.num_programs(1)
    r0 = pl.multiple_of(r * th, th)

    def phases(start, n):
        ll = jnp.maximum(ll_ref[0, pl.ds(start, n), :, :], 0.0)
        lh = lh_ref[0, pl.ds(start, n), :, :]
        hl = hl_ref[0, pl.ds(start, n), :, :]
        hh = hh_ref[0, pl.ds(start, n), :, :]
        p00 = 0.5 * (ll - lh - hl + hh)
        p01 = 0.5 * (ll + lh - hl - hh)
        p10 = 0.5 * (ll - lh + hl - hh)
        p11 = 0.5 * (ll + lh + hl + hh)
        return p00, p01, p10, p11

    m00, m01, m10, m11 = phases(r0, th)
    _, _, t10, t11 = phases(jnp.maximum(r0 - 1, 0), 1)
    b00, b01, _, _ = phases(jnp.minimum(r0 + th, H - 1), 1)
    tmask = (r > 0).astype(_f32)
    bmask = (r < nrb - 1).astype(_f32)

    zc = jnp.zeros((th + 1, 1, Cin), _f32)

    def colpad(x):
        return jnp.concatenate([zc, x, zc], axis=1).astype(_bf16)

    slabs = (
        (colpad(jnp.concatenate([m00, b00 * bmask], axis=0)),
         colpad(jnp.concatenate([m01, b01 * bmask], axis=0))),
        (colpad(jnp.concatenate([t10 * tmask, m10], axis=0)),
         colpad(jnp.concatenate([t11 * tmask, m11], axis=0))),
    )

    Cin_ = Cin
    accs = {}
    s1 = jnp.zeros((1, Cout), _f32)
    s2 = jnp.zeros((1, Cout), _f32)
    for p in (0, 1):
        for q in (0, 1):
            for k, (dy, dx) in enumerate(_TAPS):
                s = (p + dy) & 1
                t = (q + dx) & 1
                rs = (p + dy) // 2 + s
                cs = (q + dx) // 2 + 1
                a_sc[:, k * Cin_:(k + 1) * Cin_] = (
                    slabs[s][t][rs:rs + th, cs:cs + W, :].reshape(th * W, Cin_))
            acc = jnp.dot(a_sc[...], w_ref[...],
                          preferred_element_type=_f32) + b_ref[...]
            s1 = s1 + jnp.sum(acc, axis=0, keepdims=True)
            s2 = s2 + jnp.sum(acc * acc, axis=0, keepdims=True)
            accs[(p, q)] = acc.reshape(th, W, Cout)

    even = jnp.stack([accs[(0, 0)], accs[(0, 1)]], axis=2)
    odd = jnp.stack([accs[(1, 0)], accs[(1, 1)]], axis=2)
    full = jnp.stack([even, odd], axis=1)
    y_ref[0] = full.reshape(4 * th * W, Cout).astype(_bf16)

    stats_ref[0, 0, 0:1, :] = s1
    stats_ref[0, 0, 1:2, :] = s2


def _bn_apply_kernel(y_ref, scale_ref, shift_ref, o_ref):
    y = y_ref[0].astype(_f32) * scale_ref[...] + shift_ref[...]
    o_ref[0] = jnp.transpose(y, (1, 0))


def _pick_row_block(H, W, row_block):
    for th in range(min(row_block, H), 0, -1):
        if H % th == 0 and (4 * th * W) % 128 == 0:
            return th
    return H


def kernel(LL, LH, HL, HH, conv_w, conv_b, bn_gamma, bn_beta,
           *, eps=1e-5, row_block=16):
    N, Cin, H, W = LL.shape
    Cout = conv_w.shape[0]
    TH = _pick_row_block(H, W, row_block)
    R = H // TH
    L = 4 * TH * W
    OHW = 4 * H * W

    to_nhwc = lambda x: jnp.transpose(x, (0, 2, 3, 1)).astype(_f32)
    ll, lh, hl, hh = map(to_nhwc, (LL, LH, HL, HH))

    w2 = jnp.transpose(conv_w, (2, 3, 1, 0)).reshape(9 * Cin, Cout).astype(_bf16)
    b2 = conv_b.reshape(1, Cout).astype(_f32)

    band_spec = pl.BlockSpec((1, H, W, Cin), lambda n, r: (n, 0, 0, 0))

    y, stats = pl.pallas_call(
        functools.partial(_upconv_kernel, th=TH),
        out_shape=(jax.ShapeDtypeStruct((N, OHW, Cout), _bf16),
                   jax.ShapeDtypeStruct((N, R, 2, Cout), _f32)),
        grid_spec=pltpu.PrefetchScalarGridSpec(
            num_scalar_prefetch=0,
            grid=(N, R),
            in_specs=[band_spec, band_spec, band_spec, band_spec,
                      pl.BlockSpec((9 * Cin, Cout), lambda n, r: (0, 0)),
                      pl.BlockSpec((1, Cout), lambda n, r: (0, 0))],
            out_specs=(pl.BlockSpec((1, L, Cout), lambda n, r: (n, r, 0)),
                       pl.BlockSpec((1, 1, 2, Cout), lambda n, r: (n, r, 0, 0))),
            scratch_shapes=[pltpu.VMEM((TH * W, 9 * Cin), _bf16)]),
        compiler_params=pltpu.CompilerParams(
            dimension_semantics=("parallel", "parallel")),
    )(ll, lh, hl, hh, w2, b2)

    cnt = float(N * OHW)
    s = jnp.sum(stats, axis=(0, 1))
    mean = s[0] / cnt
    var = jnp.maximum(s[1] / cnt - mean * mean, 0.0)
    scale = (bn_gamma.astype(_f32) * jax.lax.rsqrt(var + eps)).reshape(1, Cout)
    shift = bn_beta.astype(_f32).reshape(1, Cout) - mean.reshape(1, Cout) * scale

    y_bn = pl.pallas_call(
        _bn_apply_kernel,
        out_shape=jax.ShapeDtypeStruct((N, Cout, OHW), _f32),
        grid_spec=pltpu.PrefetchScalarGridSpec(
            num_scalar_prefetch=0,
            grid=(N, R),
            in_specs=[pl.BlockSpec((1, L, Cout), lambda n, r: (n, r, 0)),
                      pl.BlockSpec((1, Cout), lambda n, r: (0, 0)),
                      pl.BlockSpec((1, Cout), lambda n, r: (0, 0))],
            out_specs=pl.BlockSpec((1, Cout, L), lambda n, r: (n, 0, r))),
        compiler_params=pltpu.CompilerParams(
            dimension_semantics=("parallel", "parallel")),
    )(y, scale, shift)

    return y_bn.reshape(N, Cout, 2 * H, 2 * W)
